# Initial kernel scaffold; baseline (speedup 1.0000x reference)
import jax
import jax.numpy as jnp
from jax.experimental import pallas as pl


def kernel(X, alpha, tf_prob_logits):
    raise NotImplementedError("write your pallas kernel here")


# speedup vs baseline: 9.2177x; 9.2177x over previous
import jax
import jax.numpy as jnp
from jax.experimental import pallas as pl


_ROW_BLOCK = 512


def _body(logits_ref, alpha_ref, x_ref, o_ref):
    l = logits_ref[...]
    m = jnp.max(l, axis=0, keepdims=True)
    e = jnp.exp(l - m)
    p = e / jnp.sum(e, axis=0, keepdims=True)
    c = p * alpha_ref[...]

    x = x_ref[...]
    t = jnp.tanh(x)
    s = jax.nn.sigmoid(x)
    o_ref[...] = (c[0:1, :] * x + c[1:2, :] * t
                  + c[2:3, :] * (x * x) + c[3:4, :] * s)


def kernel(X, alpha, tf_prob_logits):
    n, f = X.shape
    logits_t = tf_prob_logits.T
    alpha_r = alpha.reshape(1, f)
    grid = (n // _ROW_BLOCK,)
    return pl.pallas_call(
        _body,
        grid=grid,
        in_specs=[
            pl.BlockSpec((4, f), lambda i: (0, 0)),
            pl.BlockSpec((1, f), lambda i: (0, 0)),
            pl.BlockSpec((_ROW_BLOCK, f), lambda i: (i, 0)),
        ],
        out_specs=pl.BlockSpec((_ROW_BLOCK, f), lambda i: (i, 0)),
        out_shape=jax.ShapeDtypeStruct((n, f), X.dtype),
    )(logits_t, alpha_r, X)
